# baseline (speedup 1.0000x reference)
import jax
import jax.numpy as jnp
from jax.experimental import pallas as pl
from jax.experimental.pallas import tpu as pltpu

E = 16
TOPK = 2
RANK = 32
D = 2048
QO = 2048
VO = 512
N = 2048
SCALE = 32.0 / 32.0

BN = 256


def _fused_kernel(h_ref, rw_ref, qa_ref, qb_ref, va_ref, vb_ref,
                  q_out_ref, v_out_ref):
    h = h_ref[...]

    logits = jax.lax.dot_general(
        h, rw_ref[...], (((1,), (1,)), ((), ())),
        preferred_element_type=jnp.float32)
    eiota = jax.lax.broadcasted_iota(jnp.int32, logits.shape, 1)
    m1 = jnp.max(logits, axis=-1, keepdims=True)
    i1 = jnp.min(jnp.where(logits == m1, eiota, E), axis=-1, keepdims=True)
    masked = jnp.where(eiota == i1, -jnp.inf, logits)
    m2 = jnp.max(masked, axis=-1, keepdims=True)
    i2 = jnp.min(jnp.where(masked == m2, eiota, E), axis=-1, keepdims=True)
    z = jnp.exp(m2 - m1)
    denom = 1.0 + z
    s1 = (1.0 / denom) * SCALE
    s2 = (z / denom) * SCALE

    col_expert = jax.lax.broadcasted_iota(jnp.int32, (1, E * RANK), 1) // RANK
    w_rep = jnp.where(col_expert == i1, s1, 0.0) + jnp.where(col_expert == i2, s2, 0.0)

    hb = h.astype(jnp.bfloat16)
    q_low = jax.lax.dot_general(
        hb, qa_ref[...].astype(jnp.bfloat16), (((1,), (0,)), ((), ())),
        preferred_element_type=jnp.float32)
    q_out_ref[...] = jax.lax.dot_general(
        (q_low * w_rep).astype(jnp.bfloat16),
        qb_ref[...].astype(jnp.bfloat16), (((1,), (0,)), ((), ())),
        preferred_element_type=jnp.float32)

    v_low = jax.lax.dot_general(
        hb, va_ref[...].astype(jnp.bfloat16), (((1,), (0,)), ((), ())),
        preferred_element_type=jnp.float32)
    v_out_ref[...] = jax.lax.dot_general(
        (v_low * w_rep).astype(jnp.bfloat16),
        vb_ref[...].astype(jnp.bfloat16), (((1,), (0,)), ((), ())),
        preferred_element_type=jnp.float32)


@jax.jit
def kernel(hidden_states, router_weight, q_lora_a, q_lora_b, v_lora_a, v_lora_b):
    qa2 = q_lora_a.transpose(1, 0, 2).reshape(D, E * RANK)
    va2 = v_lora_a.transpose(1, 0, 2).reshape(D, E * RANK)
    qb2 = q_lora_b.reshape(E * RANK, QO)
    vb2 = v_lora_b.reshape(E * RANK, VO)

    grid = (N // BN,)
    q_delta, v_delta = pl.pallas_call(
        _fused_kernel,
        grid=grid,
        in_specs=[
            pl.BlockSpec((BN, D), lambda i: (i, 0)),
            pl.BlockSpec((E, D), lambda i: (0, 0)),
            pl.BlockSpec((D, E * RANK), lambda i: (0, 0)),
            pl.BlockSpec((E * RANK, QO), lambda i: (0, 0)),
            pl.BlockSpec((D, E * RANK), lambda i: (0, 0)),
            pl.BlockSpec((E * RANK, VO), lambda i: (0, 0)),
        ],
        out_specs=[
            pl.BlockSpec((BN, QO), lambda i: (i, 0)),
            pl.BlockSpec((BN, VO), lambda i: (i, 0)),
        ],
        out_shape=[
            jax.ShapeDtypeStruct((N, QO), jnp.float32),
            jax.ShapeDtypeStruct((N, VO), jnp.float32),
        ],
        compiler_params=pltpu.CompilerParams(
            dimension_semantics=("parallel",),
        ),
    )(hidden_states, router_weight, qa2, qb2, va2, vb2)
    return (q_delta, v_delta)


# speedup vs baseline: 1.0072x; 1.0072x over previous
import jax
import jax.numpy as jnp
from jax.experimental import pallas as pl
from jax.experimental.pallas import tpu as pltpu

E = 16
TOPK = 2
RANK = 32
D = 2048
QO = 2048
VO = 512
N = 2048
SCALE = 32.0 / 32.0

BN = 1024


def _fused_kernel(h_ref, rw_ref, qa_ref, qb_ref, va_ref, vb_ref,
                  q_out_ref, v_out_ref):
    h = h_ref[...]

    logits = jax.lax.dot_general(
        h, rw_ref[...], (((1,), (1,)), ((), ())),
        preferred_element_type=jnp.float32)
    eiota = jax.lax.broadcasted_iota(jnp.int32, logits.shape, 1)
    m1 = jnp.max(logits, axis=-1, keepdims=True)
    i1 = jnp.min(jnp.where(logits == m1, eiota, E), axis=-1, keepdims=True)
    masked = jnp.where(eiota == i1, -jnp.inf, logits)
    m2 = jnp.max(masked, axis=-1, keepdims=True)
    i2 = jnp.min(jnp.where(masked == m2, eiota, E), axis=-1, keepdims=True)
    z = jnp.exp(m2 - m1)
    denom = 1.0 + z
    s1 = (1.0 / denom) * SCALE
    s2 = (z / denom) * SCALE

    col_expert = jax.lax.broadcasted_iota(jnp.int32, (1, E * RANK), 1) // RANK
    w_rep = jnp.where(col_expert == i1, s1, 0.0) + jnp.where(col_expert == i2, s2, 0.0)

    hb = h.astype(jnp.bfloat16)
    q_low = jax.lax.dot_general(
        hb, qa_ref[...].astype(jnp.bfloat16), (((1,), (0,)), ((), ())),
        preferred_element_type=jnp.float32)
    q_out_ref[...] = jax.lax.dot_general(
        (q_low * w_rep).astype(jnp.bfloat16),
        qb_ref[...].astype(jnp.bfloat16), (((1,), (0,)), ((), ())),
        preferred_element_type=jnp.float32)

    v_low = jax.lax.dot_general(
        hb, va_ref[...].astype(jnp.bfloat16), (((1,), (0,)), ((), ())),
        preferred_element_type=jnp.float32)
    v_out_ref[...] = jax.lax.dot_general(
        (v_low * w_rep).astype(jnp.bfloat16),
        vb_ref[...].astype(jnp.bfloat16), (((1,), (0,)), ((), ())),
        preferred_element_type=jnp.float32)


@jax.jit
def kernel(hidden_states, router_weight, q_lora_a, q_lora_b, v_lora_a, v_lora_b):
    qa2 = q_lora_a.transpose(1, 0, 2).reshape(D, E * RANK)
    va2 = v_lora_a.transpose(1, 0, 2).reshape(D, E * RANK)
    qb2 = q_lora_b.reshape(E * RANK, QO)
    vb2 = v_lora_b.reshape(E * RANK, VO)

    grid = (N // BN,)
    q_delta, v_delta = pl.pallas_call(
        _fused_kernel,
        grid=grid,
        in_specs=[
            pl.BlockSpec((BN, D), lambda i: (i, 0)),
            pl.BlockSpec((E, D), lambda i: (0, 0)),
            pl.BlockSpec((D, E * RANK), lambda i: (0, 0)),
            pl.BlockSpec((E * RANK, QO), lambda i: (0, 0)),
            pl.BlockSpec((D, E * RANK), lambda i: (0, 0)),
            pl.BlockSpec((E * RANK, VO), lambda i: (0, 0)),
        ],
        out_specs=[
            pl.BlockSpec((BN, QO), lambda i: (i, 0)),
            pl.BlockSpec((BN, VO), lambda i: (i, 0)),
        ],
        out_shape=[
            jax.ShapeDtypeStruct((N, QO), jnp.float32),
            jax.ShapeDtypeStruct((N, VO), jnp.float32),
        ],
        compiler_params=pltpu.CompilerParams(
            dimension_semantics=("parallel",),
        ),
    )(hidden_states, router_weight, qa2, qb2, va2, vb2)
    return (q_delta, v_delta)


# speedup vs baseline: 1.0273x; 1.0200x over previous
import jax
import jax.numpy as jnp
from jax.experimental import pallas as pl
from jax.experimental.pallas import tpu as pltpu

E = 16
TOPK = 2
RANK = 32
D = 2048
QO = 2048
VO = 512
N = 2048
SCALE = 32.0 / 32.0

BN = 512


def _fused_kernel(h_ref, rw_ref, qa_ref, qb_ref, va_ref, vb_ref,
                  q_out_ref, v_out_ref):
    h = h_ref[...]

    logits = jax.lax.dot_general(
        h, rw_ref[...], (((1,), (1,)), ((), ())),
        preferred_element_type=jnp.float32)
    eiota = jax.lax.broadcasted_iota(jnp.int32, logits.shape, 1)
    m1 = jnp.max(logits, axis=-1, keepdims=True)
    i1 = jnp.min(jnp.where(logits == m1, eiota, E), axis=-1, keepdims=True)
    masked = jnp.where(eiota == i1, -jnp.inf, logits)
    m2 = jnp.max(masked, axis=-1, keepdims=True)
    i2 = jnp.min(jnp.where(masked == m2, eiota, E), axis=-1, keepdims=True)
    z = jnp.exp(m2 - m1)
    denom = 1.0 + z
    s1 = (1.0 / denom) * SCALE
    s2 = (z / denom) * SCALE

    col_expert = jax.lax.broadcasted_iota(jnp.int32, (1, E * RANK), 1) // RANK
    w_rep = jnp.where(col_expert == i1, s1, 0.0) + jnp.where(col_expert == i2, s2, 0.0)

    hb = h.astype(jnp.bfloat16)
    q_low = jax.lax.dot_general(
        hb, qa_ref[...].astype(jnp.bfloat16), (((1,), (0,)), ((), ())),
        preferred_element_type=jnp.float32)
    q_out_ref[...] = jax.lax.dot_general(
        (q_low * w_rep).astype(jnp.bfloat16),
        qb_ref[...].astype(jnp.bfloat16), (((1,), (0,)), ((), ())),
        preferred_element_type=jnp.float32)

    v_low = jax.lax.dot_general(
        hb, va_ref[...].astype(jnp.bfloat16), (((1,), (0,)), ((), ())),
        preferred_element_type=jnp.float32)
    v_out_ref[...] = jax.lax.dot_general(
        (v_low * w_rep).astype(jnp.bfloat16),
        vb_ref[...].astype(jnp.bfloat16), (((1,), (0,)), ((), ())),
        preferred_element_type=jnp.float32)


@jax.jit
def kernel(hidden_states, router_weight, q_lora_a, q_lora_b, v_lora_a, v_lora_b):
    qa2 = q_lora_a.transpose(1, 0, 2).reshape(D, E * RANK)
    va2 = v_lora_a.transpose(1, 0, 2).reshape(D, E * RANK)
    qb2 = q_lora_b.reshape(E * RANK, QO)
    vb2 = v_lora_b.reshape(E * RANK, VO)

    grid = (N // BN,)
    q_delta, v_delta = pl.pallas_call(
        _fused_kernel,
        grid=grid,
        in_specs=[
            pl.BlockSpec((BN, D), lambda i: (i, 0)),
            pl.BlockSpec((E, D), lambda i: (0, 0)),
            pl.BlockSpec((D, E * RANK), lambda i: (0, 0)),
            pl.BlockSpec((E * RANK, QO), lambda i: (0, 0)),
            pl.BlockSpec((D, E * RANK), lambda i: (0, 0)),
            pl.BlockSpec((E * RANK, VO), lambda i: (0, 0)),
        ],
        out_specs=[
            pl.BlockSpec((BN, QO), lambda i: (i, 0)),
            pl.BlockSpec((BN, VO), lambda i: (i, 0)),
        ],
        out_shape=[
            jax.ShapeDtypeStruct((N, QO), jnp.float32),
            jax.ShapeDtypeStruct((N, VO), jnp.float32),
        ],
        compiler_params=pltpu.CompilerParams(
            dimension_semantics=("parallel",),
        ),
    )(hidden_states, router_weight, qa2, qb2, va2, vb2)
    return (q_delta, v_delta)
